# baseline (speedup 1.0000x reference)
import jax
import jax.numpy as jnp
from jax import lax
from jax.experimental import pallas as pl
from jax.experimental.pallas import tpu as pltpu
from jax.experimental.pallas import tpu_sc as plsc

D = 64
SCALE = 8.0
NC, NS = 2, 16
NW = NC * NS
BATCH = 4096
TOK = 200
VIEW_ROWS = 500_000
CHUNK = 128
NT = TOK


VFULL = 999_936
NBLK = VFULL // 128
BLK_BASE = NBLK // NW


def _tbody(tblt_hbm, tail_hbm, out_hbm, s0, s1, t0, t1, chan_v,
           is0, is1, ts0, ts1):
    sbuf = (s0, s1)
    tbuf = (t0, t1)
    isem = (is0, is1)
    tsem = (ts0, ts1)
    c = lax.axis_index("c")
    s = lax.axis_index("s")
    w = c * NS + s
    nblk = BLK_BASE + jnp.where(w < NBLK - BLK_BASE * NW, 1, 0)
    k0 = w * BLK_BASE + jnp.minimum(w, NBLK - BLK_BASE * NW)

    iota = lax.iota(jnp.int32, 16)

    def start_in(k, b):
        pltpu.async_copy(tblt_hbm.at[:, pl.ds(k * 128, 128)], sbuf[b],
                         isem[b])

    def wait_in(k, b):
        pltpu.make_async_copy(tblt_hbm.at[:, pl.ds(k * 128, 128)], sbuf[b],
                              isem[b]).wait()

    def start_outc(k, b):
        pltpu.async_copy(tbuf[b], out_hbm.at[pl.ds(k * 64, 64)], tsem[b])

    def wait_outc(k, b):
        pltpu.make_async_copy(tbuf[b], out_hbm.at[pl.ds(k * 64, 64)],
                              tsem[b]).wait()

    for col in range(D):
        chan_v[pl.ds(col * 16, 16)] = (col + iota) & (D - 1)

    def transpose(b):
        def grp(gi, carry):
            vloc = gi * 16 + iota
            rowd = lax.shift_right_logical(vloc, 1)
            colbase = (vloc & 1) * D
            for col in range(D):
                sl = pl.ds(col * 16, 16)
                cv = chan_v[sl]
                v = plsc.load_gather(sbuf[b], [cv, vloc])
                plsc.store_scatter(tbuf[b], [rowd, colbase + cv], v)
            return carry
        lax.fori_loop(0, 8, grp, 0)

    start_in(k0, 0)

    def step(g, carry):
        for b in range(2):
            i = g * 2 + b
            k = k0 + i

            @pl.when(i + 1 < nblk)
            def _():
                start_in(k + 1, 1 - b)

            @pl.when(i < nblk)
            def _():
                wait_in(k, b)

                @pl.when(i >= 2)
                def _():
                    wait_outc(k - 2, b)

                transpose(b)
                start_outc(k, b)
        return carry

    lax.fori_loop(0, (BLK_BASE + 2) // 2, step, 0, unroll=False)

    wait_outc(k0, 0)
    wait_outc(k0, 1)

    @pl.when(w == 0)
    def _():
        pltpu.sync_copy(tail_hbm, sbuf[0].at[:32, :])
        pltpu.sync_copy(sbuf[0].at[:32, :], out_hbm.at[pl.ds(VFULL // 2, 32)])


def _body(ids_hbm, tbl_hbm, out_hbm, ids_v, chan_v, x0, x1,
          g0, g1, o0, o1, gs0, gs1, os0, os1):
    idx2 = (x0, x1)
    gbuf = (g0, g1)
    obuf = (o0, o1)
    gsem = (gs0, gs1)
    osem = (os0, os1)
    c = lax.axis_index("c")
    s = lax.axis_index("s")
    w = c * NS + s
    col0 = w * CHUNK

    pltpu.sync_copy(ids_hbm.at[:, pl.ds(col0, CHUNK)], ids_v)

    def prep_gather(j, b):
        def grp(gi, carry):
            sl = pl.ds(gi * 16, 16)
            idx2[b][sl] = lax.shift_right_logical(ids_v[j, sl], 1)
            return carry
        lax.fori_loop(0, CHUNK // 16, grp, 0)
        pltpu.async_copy(tbl_hbm.at[idx2[b]], gbuf[b], gsem[b])

    def wait_gather(j, b):
        pltpu.make_async_copy(tbl_hbm.at[idx2[b]], gbuf[b], gsem[b]).wait()

    def start_out(j, b):
        pltpu.async_copy(obuf[b], out_hbm.at[j, :, pl.ds(col0, CHUNK)],
                         osem[b])

    def wait_out(j, b):
        pltpu.make_async_copy(obuf[b], out_hbm.at[j, :, pl.ds(col0, CHUNK)],
                              osem[b]).wait()

    iota = lax.iota(jnp.int32, 16)

    for col in range(D):
        chan_v[pl.ds(col * 16, 16)] = (col + iota) & (D - 1)

    def transpose_scale(j, b):
        def grp(gi, carry):
            sl = pl.ds(gi * 16, 16)
            idsv = ids_v[j, sl]
            colbase = (idsv & 1) * D
            mv = jnp.where(idsv != 0, jnp.float32(SCALE), jnp.float32(0.0))
            rows = gi * 16 + iota
            for col in range(D):
                csl = pl.ds(col * 16, 16)
                cv = chan_v[csl]
                v = plsc.load_gather(gbuf[b], [rows, colbase + cv])
                plsc.store_scatter(obuf[b], [cv, rows], v * mv)
            return carry
        lax.fori_loop(0, CHUNK // 16, grp, 0)

    prep_gather(0, 0)

    def step(g, carry):
        for b in range(2):
            j = g * 2 + b

            @pl.when(j + 1 < NT)
            def _():
                prep_gather(j + 1, 1 - b)

            wait_gather(j, b)

            @pl.when(j >= 2)
            def _():
                wait_out(j - 2, b)

            transpose_scale(j, b)
            start_out(j, b)
        return carry

    lax.fori_loop(0, NT // 2, step, 0, unroll=False)
    wait_out(NT - 2, 0)
    wait_out(NT - 1, 1)


@jax.jit
def _run1(tblt, tail):
    mesh = plsc.VectorSubcoreMesh(core_axis_name="c", subcore_axis_name="s")
    f = pl.kernel(
        _tbody,
        out_type=jax.ShapeDtypeStruct((VIEW_ROWS, 2 * D), jnp.float32),
        mesh=mesh,
        compiler_params=pltpu.CompilerParams(needs_layout_passes=False,
                                             use_tc_tiling_on_sc=True),
        scratch_types=(
            [pltpu.VMEM((D, CHUNK), jnp.float32)] * 4
            + [pltpu.VMEM((D * 16,), jnp.int32)]
            + [pltpu.SemaphoreType.DMA] * 4
        ),
    )
    return f(tblt, tail)


@jax.jit
def _run2(ids_t, tbl_view):
    mesh = plsc.VectorSubcoreMesh(core_axis_name="c", subcore_axis_name="s")
    f = pl.kernel(
        _body,
        out_type=jax.ShapeDtypeStruct((TOK, D, BATCH), jnp.float32),
        mesh=mesh,
        compiler_params=pltpu.CompilerParams(needs_layout_passes=False,
                                             use_tc_tiling_on_sc=True),
        scratch_types=(
            [pltpu.VMEM((NT, CHUNK), jnp.int32)]
            + [pltpu.VMEM((D * 16,), jnp.int32)]
            + [pltpu.VMEM((CHUNK,), jnp.int32)] * 2
            + [pltpu.VMEM((CHUNK, 2 * D), jnp.float32)] * 2
            + [pltpu.VMEM((D, CHUNK), jnp.float32)] * 2
            + [pltpu.SemaphoreType.DMA] * 4
        ),
    )
    return f(ids_t, tbl_view)


def kernel(input, lookup_table):
    ids_t = input.astype(jnp.int32).T
    tblt = lookup_table.T
    tail = lookup_table[VFULL:].reshape(32, 2 * D)
    tbl_view = _run1(tblt, tail)
    out_p = _run2(ids_t, tbl_view)
    return out_p.transpose(2, 0, 1)


# speedup vs baseline: 2.0026x; 2.0026x over previous
import jax
import jax.numpy as jnp
from jax import lax
from jax.experimental import pallas as pl
from jax.experimental.pallas import tpu as pltpu
from jax.experimental.pallas import tpu_sc as plsc

D = 64
SCALE = 8.0
NC, NS = 2, 16
NW = NC * NS
BATCH = 4096
TOK = 200
VIEW_ROWS = 500_000
CHUNK = 128
NT = TOK


def _body(ids_hbm, tbl_hbm, out_hbm, ids_v, x0, x1,
          g0, g1, o0, o1, gs0, gs1, os0, os1):
    idx2 = (x0, x1)
    gbuf = (g0, g1)
    obuf = (o0, o1)
    gsem = (gs0, gs1)
    osem = (os0, os1)
    c = lax.axis_index("c")
    s = lax.axis_index("s")
    w = c * NS + s
    col0 = w * CHUNK

    pltpu.sync_copy(ids_hbm.at[:, pl.ds(col0, CHUNK)], ids_v)

    def prep_gather(j, b):
        def grp(gi, carry):
            sl = pl.ds(gi * 16, 16)
            idx2[b][sl] = lax.shift_right_logical(ids_v[j, sl], 1)
            return carry
        lax.fori_loop(0, CHUNK // 16, grp, 0)
        pltpu.async_copy(tbl_hbm.at[idx2[b]], gbuf[b], gsem[b])

    def wait_gather(j, b):
        pltpu.make_async_copy(tbl_hbm.at[idx2[b]], gbuf[b], gsem[b]).wait()

    def start_out(j, b):
        pltpu.async_copy(obuf[b], out_hbm.at[j, :, pl.ds(col0, CHUNK)],
                         osem[b])

    def wait_out(j, b):
        pltpu.make_async_copy(obuf[b], out_hbm.at[j, :, pl.ds(col0, CHUNK)],
                              osem[b]).wait()

    iota = lax.iota(jnp.int32, 16)

    def transpose_scale(j, b):
        def grp(gi, carry):
            sl = pl.ds(gi * 16, 16)
            idsv = ids_v[j, sl]
            colbase = (idsv & 1) * D
            mv = jnp.where(idsv != 0, jnp.float32(SCALE), jnp.float32(0.0))
            rows = gi * 16 + iota
            for col in range(D):
                chan = (col + iota) & (D - 1)
                v = plsc.load_gather(gbuf[b], [rows, colbase + chan])
                plsc.store_scatter(obuf[b], [chan, rows], v * mv)
            return carry
        lax.fori_loop(0, CHUNK // 16, grp, 0)

    prep_gather(0, 0)

    def step(g, carry):
        for b in range(2):
            j = g * 2 + b

            @pl.when(j + 1 < NT)
            def _():
                prep_gather(j + 1, 1 - b)

            wait_gather(j, b)

            @pl.when(j >= 2)
            def _():
                wait_out(j - 2, b)

            transpose_scale(j, b)
            start_out(j, b)
        return carry

    lax.fori_loop(0, NT // 2, step, 0, unroll=False)
    wait_out(NT - 2, 0)
    wait_out(NT - 1, 1)


@jax.jit
def _run2(ids_t, tbl_view):
    mesh = plsc.VectorSubcoreMesh(core_axis_name="c", subcore_axis_name="s")
    f = pl.kernel(
        _body,
        out_type=jax.ShapeDtypeStruct((TOK, D, BATCH), jnp.float32),
        mesh=mesh,
        compiler_params=pltpu.CompilerParams(needs_layout_passes=False,
                                             use_tc_tiling_on_sc=True),
        scratch_types=(
            [pltpu.VMEM((NT, CHUNK), jnp.int32)]
            + [pltpu.VMEM((CHUNK,), jnp.int32)] * 2
            + [pltpu.VMEM((CHUNK, 2 * D), jnp.float32)] * 2
            + [pltpu.VMEM((D, CHUNK), jnp.float32)] * 2
            + [pltpu.SemaphoreType.DMA] * 4
        ),
    )
    return f(ids_t, tbl_view)


def kernel(input, lookup_table):
    ids_t = input.astype(jnp.int32).T
    tbl_view = lookup_table.reshape(VIEW_ROWS, 2 * D)
    out_p = _run2(ids_t, tbl_view)
    return out_p.transpose(2, 0, 1)
